# baseline (speedup 1.0000x reference)
import functools

import jax
import jax.numpy as jnp
from jax import lax
from jax.experimental import pallas as pl
from jax.experimental.pallas import tpu as pltpu
from jax.experimental.pallas import tpu_sc as plsc

N_NODES = 10000
N_EDGES = 320000
D = 128

NC = 2
NS = 16
NW = NC * NS
N_PAD = 10240
E_PAD = 327680
EPW = E_PAD // NW
K = 80
NCHUNK = EPW // K
RPT = N_PAD // NS


DEG_R = N_PAD // 128


def _deg_kernel():
    mesh = plsc.VectorSubcoreMesh(core_axis_name="c", subcore_axis_name="s")

    @functools.partial(
        pl.kernel,
        mesh=mesh,
        out_type=jax.ShapeDtypeStruct((NC, N_PAD, D), jnp.float32),
        scratch_types=[
            pltpu.VMEM((K,), jnp.int32),
            pltpu.VMEM((K, D), jnp.float32),
            pltpu.VMEM_SHARED((N_PAD, D), jnp.float32),
        ],
    )
    def k(dst_hbm, ones_hbm, zeros_hbm, out_hbm, dst_v, ones_v, acc):
        c = lax.axis_index("c")
        s = lax.axis_index("s")
        wid = c * NS + s
        r0 = s * RPT
        pltpu.sync_copy(zeros_hbm, acc.at[pl.ds(r0, RPT)])
        pltpu.sync_copy(ones_hbm, ones_v)
        plsc.subcore_barrier()
        base0 = wid * EPW

        def body(j, carry):
            pltpu.sync_copy(dst_hbm.at[pl.ds(base0 + j * K, K)], dst_v)
            pltpu.sync_copy(ones_v, acc.at[dst_v], add=True)
            return carry

        lax.fori_loop(0, NCHUNK, body, 0)
        plsc.subcore_barrier()
        pltpu.sync_copy(acc.at[pl.ds(r0, RPT)],
                        out_hbm.at[c].at[pl.ds(r0, RPT)])

    return k


NB = 4
GD = 2
C0 = 2 * NCHUNK


def _scatter_kernel():
    mesh = plsc.VectorSubcoreMesh(core_axis_name="c", subcore_axis_name="s")

    @functools.partial(
        pl.kernel,
        mesh=mesh,
        out_type=jax.ShapeDtypeStruct((N_PAD, D), jnp.float32),
        scratch_types=(
            [pltpu.VMEM((K,), jnp.int32)] * NB
            + [pltpu.VMEM((K,), jnp.int32)] * NB
            + [pltpu.VMEM((K, D), jnp.float32)] * NB
            + [pltpu.VMEM_SHARED((N_PAD, D), jnp.float32),
               pltpu.SemaphoreType.DMA((NB,)),
               pltpu.SemaphoreType.DMA((NB,)),
               pltpu.SemaphoreType.DMA((NB,))]
        ),
    )
    def k(hs_hbm, src_hbm, dst_hbm, zeros_hbm, out_hbm, *scr):
        srcv = scr[0:NB]
        dstv = scr[NB:2 * NB]
        rows = scr[2 * NB:3 * NB]
        acc, semIs, semId, semG = scr[3 * NB:]
        c = lax.axis_index("c")
        s = lax.axis_index("s")
        r0 = s * RPT
        base0 = s * (C0 * K)

        def idx_issue(j, b):
            off = pl.multiple_of(base0 + j * K, 8)
            pltpu.async_copy(src_hbm.at[pl.ds(off, K)], srcv[b],
                             semIs.at[b])
            pltpu.async_copy(dst_hbm.at[pl.ds(off, K)], dstv[b],
                             semId.at[b])

        def idx_wait_src(j, b):
            off = pl.multiple_of(base0 + j * K, 8)
            pltpu.make_async_copy(src_hbm.at[pl.ds(off, K)], srcv[b],
                                  semIs.at[b]).wait()

        def idx_wait_dst(j, b):
            off = pl.multiple_of(base0 + j * K, 8)
            pltpu.make_async_copy(dst_hbm.at[pl.ds(off, K)], dstv[b],
                                  semId.at[b]).wait()

        @pl.when(c == 0)
        def _():
            for b in range(NB):
                idx_issue(b, b)
            pltpu.sync_copy(zeros_hbm, acc.at[pl.ds(r0, RPT)])
            for j in range(GD):
                idx_wait_src(j, j % NB)
                pltpu.async_copy(hs_hbm.at[srcv[j % NB]], rows[j % NB],
                                 semG.at[j % NB])
            plsc.subcore_barrier()

            def body(g, carry):
                for b in range(NB):
                    j = g * NB + b
                    bg = (b + GD) % NB

                    @pl.when(j + GD < C0)
                    def _():
                        idx_wait_src(j + GD, bg)
                        pltpu.async_copy(hs_hbm.at[srcv[bg]], rows[bg],
                                         semG.at[bg])

                    pltpu.make_async_copy(hs_hbm.at[srcv[b]], rows[b],
                                          semG.at[b]).wait()
                    idx_wait_dst(j, b)
                    pltpu.sync_copy(rows[b], acc.at[dstv[b]], add=True)

                    @pl.when(j + NB < C0)
                    def _():
                        idx_issue(j + NB, b)
                return carry

            lax.fori_loop(0, C0 // NB, body, 0)
            plsc.subcore_barrier()
            pltpu.sync_copy(acc.at[pl.ds(r0, RPT)],
                            out_hbm.at[pl.ds(r0, RPT)])

    return k


_ROWS = 2000


def _tc1_body(x_ref, w_ref, degp_ref, hs_ref, dinv_ref):
    d = lax.rsqrt(degp_ref[0, :, 0:1] + degp_ref[1, :, 0:1] + 1.0)
    dinv_ref[...] = d
    h = jnp.dot(x_ref[...], w_ref[...], preferred_element_type=jnp.float32)
    hs_ref[...] = h * d


def _tc2_body(agg_ref, hs_ref, dinv_ref, b_ref, w_ref, out_ref):
    d = dinv_ref[...]
    t = (agg_ref[...] + hs_ref[...]) * d + b_ref[...]
    h1 = jnp.maximum(t, 0.0)
    out_ref[...] = jnp.dot(h1, w_ref[...],
                           preferred_element_type=jnp.float32) * d


def _tc3_body(agg_ref, hs_ref, dinv_ref, b_ref, out_ref):
    d = dinv_ref[...]
    out_ref[...] = (agg_ref[...] + hs_ref[...]) * d + b_ref[...]


def _row_spec(width):
    return pl.BlockSpec((_ROWS, width), lambda i: (i, 0))


def _part_spec(width):
    return pl.BlockSpec((NC, _ROWS, width), lambda i: (0, i, 0))


def _full_spec(r, c):
    return pl.BlockSpec((r, c), lambda i: (0, 0))


_GRID = N_NODES // _ROWS


def _tc1(x, w1, degp):
    return pl.pallas_call(
        _tc1_body,
        grid=(_GRID,),
        in_specs=[_row_spec(D), _full_spec(D, D), _part_spec(D)],
        out_specs=[_row_spec(D), _row_spec(1)],
        out_shape=[jax.ShapeDtypeStruct((N_NODES, D), jnp.float32),
                   jax.ShapeDtypeStruct((N_NODES, 1), jnp.float32)],
    )(x, w1, degp)


def _tc2(aggp, hs, dinv, b, w2):
    return pl.pallas_call(
        _tc2_body,
        grid=(_GRID,),
        in_specs=[_row_spec(D), _row_spec(D), _row_spec(1),
                  _full_spec(1, D), _full_spec(D, D)],
        out_specs=_row_spec(D),
        out_shape=jax.ShapeDtypeStruct((N_NODES, D), jnp.float32),
    )(aggp, hs, dinv, b, w2)


def _tc3(aggp, hs, dinv, b):
    return pl.pallas_call(
        _tc3_body,
        grid=(_GRID,),
        in_specs=[_row_spec(D), _row_spec(D), _row_spec(1),
                  _full_spec(1, D)],
        out_specs=_row_spec(D),
        out_shape=jax.ShapeDtypeStruct((N_NODES, D), jnp.float32),
    )(aggp, hs, dinv, b)


_deg = _deg_kernel()
_scatter = _scatter_kernel()


@jax.jit
def kernel(x, edge_index, W1, b1, W2, b2):
    ei = edge_index.astype(jnp.int32)
    npad = E_PAD - N_EDGES
    src = jnp.concatenate([ei[0], jnp.zeros((npad,), jnp.int32)])
    pad_dst = N_NODES + (jnp.arange(npad, dtype=jnp.int32)
                         % (N_PAD - N_NODES))
    dst = jnp.concatenate([ei[1], pad_dst])
    ones_deg = jnp.ones((K, D), jnp.float32)
    zeros_rows = jnp.zeros((RPT, D), jnp.float32)

    degp = _deg(dst, ones_deg, zeros_rows)[:, :N_NODES]
    hs1, dinv = _tc1(x, W1, degp)
    agg1 = _scatter(hs1, src, dst, zeros_rows)[:N_NODES]
    hs2 = _tc2(agg1, hs1, dinv, b1.reshape(1, D), W2)
    agg2 = _scatter(hs2, src, dst, zeros_rows)[:N_NODES]
    return _tc3(agg2, hs2, dinv, b2.reshape(1, D))


# speedup vs baseline: 3.2586x; 3.2586x over previous
import functools

import jax
import jax.numpy as jnp
from jax import lax
from jax.experimental import pallas as pl
from jax.experimental.pallas import tpu as pltpu
from jax.experimental.pallas import tpu_sc as plsc

N_NODES = 10000
N_EDGES = 320000
D = 128

NC = 2
NS = 16
NW = NC * NS
N_PAD = 10240
E_PAD = 327680
EPW = E_PAD // NW
K = 80
NCHUNK = EPW // K
RPT = N_PAD // NS


DEG_R = N_PAD // 128


def _deg_kernel():
    mesh = plsc.VectorSubcoreMesh(core_axis_name="c", subcore_axis_name="s")

    @functools.partial(
        pl.kernel,
        mesh=mesh,
        out_type=jax.ShapeDtypeStruct((NC, N_PAD, D), jnp.float32),
        scratch_types=[
            pltpu.VMEM((K,), jnp.int32),
            pltpu.VMEM((K, D), jnp.float32),
            pltpu.VMEM_SHARED((N_PAD, D), jnp.float32),
        ],
    )
    def k(dst_hbm, ones_hbm, zeros_hbm, out_hbm, dst_v, ones_v, acc):
        c = lax.axis_index("c")
        s = lax.axis_index("s")
        wid = c * NS + s
        r0 = s * RPT
        pltpu.sync_copy(zeros_hbm, acc.at[pl.ds(r0, RPT)])
        pltpu.sync_copy(ones_hbm, ones_v)
        plsc.subcore_barrier()
        base0 = wid * EPW

        def body(j, carry):
            pltpu.sync_copy(dst_hbm.at[pl.ds(base0 + j * K, K)], dst_v)
            pltpu.sync_copy(ones_v, acc.at[dst_v], add=True)
            return carry

        lax.fori_loop(0, NCHUNK, body, 0)
        plsc.subcore_barrier()
        pltpu.sync_copy(acc.at[pl.ds(r0, RPT)],
                        out_hbm.at[c].at[pl.ds(r0, RPT)])

    return k


NB = 4
GD = 2


def _scatter_kernel():
    mesh = plsc.VectorSubcoreMesh(core_axis_name="c", subcore_axis_name="s")

    @functools.partial(
        pl.kernel,
        mesh=mesh,
        out_type=jax.ShapeDtypeStruct((NC, N_PAD, D), jnp.float32),
        scratch_types=(
            [pltpu.VMEM((K,), jnp.int32)] * NB
            + [pltpu.VMEM((K,), jnp.int32)] * NB
            + [pltpu.VMEM((K, D), jnp.float32)] * NB
            + [pltpu.VMEM_SHARED((N_PAD, D), jnp.float32),
               pltpu.SemaphoreType.DMA((NB,)),
               pltpu.SemaphoreType.DMA((NB,)),
               pltpu.SemaphoreType.DMA((NB,))]
        ),
    )
    def k(hs_hbm, src_hbm, dst_hbm, zeros_hbm, out_hbm, *scr):
        srcv = scr[0:NB]
        dstv = scr[NB:2 * NB]
        rows = scr[2 * NB:3 * NB]
        acc, semIs, semId, semG = scr[3 * NB:]
        c = lax.axis_index("c")
        s = lax.axis_index("s")
        wid = c * NS + s
        r0 = s * RPT
        base0 = wid * EPW

        def idx_issue(j, b):
            off = pl.multiple_of(base0 + j * K, 8)
            pltpu.async_copy(src_hbm.at[pl.ds(off, K)], srcv[b],
                             semIs.at[b])
            pltpu.async_copy(dst_hbm.at[pl.ds(off, K)], dstv[b],
                             semId.at[b])

        def idx_wait_src(j, b):
            off = pl.multiple_of(base0 + j * K, 8)
            pltpu.make_async_copy(src_hbm.at[pl.ds(off, K)], srcv[b],
                                  semIs.at[b]).wait()

        def idx_wait_dst(j, b):
            off = pl.multiple_of(base0 + j * K, 8)
            pltpu.make_async_copy(dst_hbm.at[pl.ds(off, K)], dstv[b],
                                  semId.at[b]).wait()

        for b in range(NB):
            idx_issue(b, b)
        pltpu.sync_copy(zeros_hbm, acc.at[pl.ds(r0, RPT)])
        for j in range(GD):
            idx_wait_src(j, j % NB)
            pltpu.async_copy(hs_hbm.at[srcv[j % NB]], rows[j % NB],
                             semG.at[j % NB])
        plsc.subcore_barrier()

        def body(g, carry):
            for b in range(NB):
                j = g * NB + b
                bg = (b + GD) % NB

                @pl.when(j + GD < NCHUNK)
                def _():
                    idx_wait_src(j + GD, bg)
                    pltpu.async_copy(hs_hbm.at[srcv[bg]], rows[bg],
                                     semG.at[bg])

                pltpu.make_async_copy(hs_hbm.at[srcv[b]], rows[b],
                                      semG.at[b]).wait()
                idx_wait_dst(j, b)
                pltpu.sync_copy(rows[b], acc.at[dstv[b]], add=True)

                @pl.when(j + NB < NCHUNK)
                def _():
                    idx_issue(j + NB, b)
            return carry

        lax.fori_loop(0, NCHUNK // NB, body, 0)
        plsc.subcore_barrier()
        pltpu.sync_copy(acc.at[pl.ds(r0, RPT)],
                        out_hbm.at[c].at[pl.ds(r0, RPT)])

    return k


_ROWS = 2000


def _tc1_body(x_ref, w_ref, degp_ref, hs_ref, dinv_ref):
    d = lax.rsqrt(degp_ref[0, :, 0:1] + degp_ref[1, :, 0:1] + 1.0)
    dinv_ref[...] = d
    h = jnp.dot(x_ref[...], w_ref[...], preferred_element_type=jnp.float32)
    hs_ref[...] = h * d


def _tc2_body(aggp_ref, hs_ref, dinv_ref, b_ref, w_ref, out_ref):
    d = dinv_ref[...]
    t = (aggp_ref[0] + aggp_ref[1] + hs_ref[...]) * d + b_ref[...]
    h1 = jnp.maximum(t, 0.0)
    out_ref[...] = jnp.dot(h1, w_ref[...],
                           preferred_element_type=jnp.float32) * d


def _tc3_body(aggp_ref, hs_ref, dinv_ref, b_ref, out_ref):
    d = dinv_ref[...]
    out_ref[...] = (aggp_ref[0] + aggp_ref[1] + hs_ref[...]) * d + b_ref[...]


def _row_spec(width):
    return pl.BlockSpec((_ROWS, width), lambda i: (i, 0))


def _part_spec(width):
    return pl.BlockSpec((NC, _ROWS, width), lambda i: (0, i, 0))


def _full_spec(r, c):
    return pl.BlockSpec((r, c), lambda i: (0, 0))


_GRID = N_NODES // _ROWS


def _tc1(x, w1, degp):
    return pl.pallas_call(
        _tc1_body,
        grid=(_GRID,),
        in_specs=[_row_spec(D), _full_spec(D, D), _part_spec(D)],
        out_specs=[_row_spec(D), _row_spec(1)],
        out_shape=[jax.ShapeDtypeStruct((N_NODES, D), jnp.float32),
                   jax.ShapeDtypeStruct((N_NODES, 1), jnp.float32)],
    )(x, w1, degp)


def _tc2(aggp, hs, dinv, b, w2):
    return pl.pallas_call(
        _tc2_body,
        grid=(_GRID,),
        in_specs=[_part_spec(D), _row_spec(D), _row_spec(1),
                  _full_spec(1, D), _full_spec(D, D)],
        out_specs=_row_spec(D),
        out_shape=jax.ShapeDtypeStruct((N_NODES, D), jnp.float32),
    )(aggp, hs, dinv, b, w2)


def _tc3(aggp, hs, dinv, b):
    return pl.pallas_call(
        _tc3_body,
        grid=(_GRID,),
        in_specs=[_part_spec(D), _row_spec(D), _row_spec(1),
                  _full_spec(1, D)],
        out_specs=_row_spec(D),
        out_shape=jax.ShapeDtypeStruct((N_NODES, D), jnp.float32),
    )(aggp, hs, dinv, b)


_deg = _deg_kernel()
_scatter = _scatter_kernel()


@jax.jit
def kernel(x, edge_index, W1, b1, W2, b2):
    ei = edge_index.astype(jnp.int32)
    npad = E_PAD - N_EDGES
    pad_src = jnp.arange(npad, dtype=jnp.int32) % N_NODES
    pad_dst = N_NODES + (jnp.arange(npad, dtype=jnp.int32)
                         % (N_PAD - N_NODES))
    src = jnp.concatenate([ei[0], pad_src])
    dst = jnp.concatenate([ei[1], pad_dst])
    ones_deg = jnp.ones((K, D), jnp.float32)
    zeros_rows = jnp.zeros((RPT, D), jnp.float32)

    degp = _deg(dst, ones_deg, zeros_rows)[:, :N_NODES]
    hs1, dinv = _tc1(x, W1, degp)
    agg1 = _scatter(hs1, src, dst, zeros_rows)[:, :N_NODES]
    hs2 = _tc2(agg1, hs1, dinv, b1.reshape(1, D), W2)
    agg2 = _scatter(hs2, src, dst, zeros_rows)[:, :N_NODES]
    return _tc3(agg2, hs2, dinv, b2.reshape(1, D))


# speedup vs baseline: 3.7616x; 1.1543x over previous
import functools

import jax
import jax.numpy as jnp
from jax import lax
from jax.experimental import pallas as pl
from jax.experimental.pallas import tpu as pltpu
from jax.experimental.pallas import tpu_sc as plsc

N_NODES = 10000
N_EDGES = 320000
D = 128

NC = 2
NS = 16
NW = NC * NS
N_PAD = 10240
E_PAD = 327680
EPW = E_PAD // NW
K = 80
NCHUNK = EPW // K
RPT = N_PAD // NS


DEG_R = N_PAD // 128


def _deg_kernel():
    mesh = plsc.VectorSubcoreMesh(core_axis_name="c", subcore_axis_name="s")

    DNB = 8
    DPD = 4

    @functools.partial(
        pl.kernel,
        mesh=mesh,
        out_type=jax.ShapeDtypeStruct((NC, N_PAD, D), jnp.float32),
        scratch_types=(
            [pltpu.VMEM((K,), jnp.int32)] * DNB
            + [pltpu.VMEM((K, D), jnp.float32),
               pltpu.VMEM_SHARED((N_PAD, D), jnp.float32),
               pltpu.SemaphoreType.DMA((DNB,)),
               pltpu.SemaphoreType.DMA((DNB,))]
        ),
    )
    def k(dst_hbm, ones_hbm, zeros_hbm, out_hbm, *scr):
        dstv = scr[0:DNB]
        ones_v, acc, semId, semS = scr[DNB:]
        c = lax.axis_index("c")
        s = lax.axis_index("s")
        wid = c * NS + s
        r0 = s * RPT
        base0 = wid * EPW

        def idx_issue(j, b):
            off = pl.multiple_of(base0 + j * K, 8)
            pltpu.async_copy(dst_hbm.at[pl.ds(off, K)], dstv[b],
                             semId.at[b])

        def idx_wait(j, b):
            off = pl.multiple_of(base0 + j * K, 8)
            pltpu.make_async_copy(dst_hbm.at[pl.ds(off, K)], dstv[b],
                                  semId.at[b]).wait()

        def sca_wait(b):
            pltpu.make_async_copy(ones_v, acc.at[dstv[b]],
                                  semS.at[b]).wait()

        for j in range(DPD):
            idx_issue(j, j % DNB)
        pltpu.sync_copy(zeros_hbm, acc.at[pl.ds(r0, RPT)])
        pltpu.sync_copy(ones_hbm, ones_v)
        plsc.subcore_barrier()

        def body(g, carry):
            for b in range(DNB):
                i = g * DNB + b
                idx_wait(i, b)
                pltpu.async_copy(ones_v, acc.at[dstv[b]], semS.at[b],
                                 add=True)
                t = i + DPD
                bt = (b + DPD) % DNB

                @pl.when(t < NCHUNK)
                def _():
                    @pl.when(i >= DPD)
                    def _():
                        sca_wait(bt)

                    idx_issue(t, bt)
            return carry

        lax.fori_loop(0, NCHUNK // DNB, body, 0)
        for b in range(DNB):
            sca_wait(b)
        plsc.subcore_barrier()
        pltpu.sync_copy(acc.at[pl.ds(r0, RPT)],
                        out_hbm.at[c].at[pl.ds(r0, RPT)])

    return k


NB = 4
GD = 2


def _scatter_kernel():
    mesh = plsc.VectorSubcoreMesh(core_axis_name="c", subcore_axis_name="s")

    @functools.partial(
        pl.kernel,
        mesh=mesh,
        out_type=jax.ShapeDtypeStruct((NC, N_PAD, D), jnp.float32),
        scratch_types=(
            [pltpu.VMEM((K,), jnp.int32)] * NB
            + [pltpu.VMEM((K,), jnp.int32)] * NB
            + [pltpu.VMEM((K, D), jnp.float32)] * NB
            + [pltpu.VMEM_SHARED((N_PAD, D), jnp.float32),
               pltpu.SemaphoreType.DMA((NB,)),
               pltpu.SemaphoreType.DMA((NB,)),
               pltpu.SemaphoreType.DMA((NB,))]
        ),
    )
    def k(hs_hbm, src_hbm, dst_hbm, zeros_hbm, out_hbm, *scr):
        srcv = scr[0:NB]
        dstv = scr[NB:2 * NB]
        rows = scr[2 * NB:3 * NB]
        acc, semIs, semId, semG = scr[3 * NB:]
        c = lax.axis_index("c")
        s = lax.axis_index("s")
        wid = c * NS + s
        r0 = s * RPT
        base0 = wid * EPW

        def idx_issue(j, b):
            off = pl.multiple_of(base0 + j * K, 8)
            pltpu.async_copy(src_hbm.at[pl.ds(off, K)], srcv[b],
                             semIs.at[b])
            pltpu.async_copy(dst_hbm.at[pl.ds(off, K)], dstv[b],
                             semId.at[b])

        def idx_wait_src(j, b):
            off = pl.multiple_of(base0 + j * K, 8)
            pltpu.make_async_copy(src_hbm.at[pl.ds(off, K)], srcv[b],
                                  semIs.at[b]).wait()

        def idx_wait_dst(j, b):
            off = pl.multiple_of(base0 + j * K, 8)
            pltpu.make_async_copy(dst_hbm.at[pl.ds(off, K)], dstv[b],
                                  semId.at[b]).wait()

        for b in range(NB):
            idx_issue(b, b)
        pltpu.sync_copy(zeros_hbm, acc.at[pl.ds(r0, RPT)])
        for j in range(GD):
            idx_wait_src(j, j % NB)
            pltpu.async_copy(hs_hbm.at[srcv[j % NB]], rows[j % NB],
                             semG.at[j % NB])
        plsc.subcore_barrier()

        def body(g, carry):
            for b in range(NB):
                j = g * NB + b
                bg = (b + GD) % NB

                @pl.when(j + GD < NCHUNK)
                def _():
                    idx_wait_src(j + GD, bg)
                    pltpu.async_copy(hs_hbm.at[srcv[bg]], rows[bg],
                                     semG.at[bg])

                pltpu.make_async_copy(hs_hbm.at[srcv[b]], rows[b],
                                      semG.at[b]).wait()
                idx_wait_dst(j, b)
                pltpu.sync_copy(rows[b], acc.at[dstv[b]], add=True)

                @pl.when(j + NB < NCHUNK)
                def _():
                    idx_issue(j + NB, b)
            return carry

        lax.fori_loop(0, NCHUNK // NB, body, 0)
        plsc.subcore_barrier()
        pltpu.sync_copy(acc.at[pl.ds(r0, RPT)],
                        out_hbm.at[c].at[pl.ds(r0, RPT)])

    return k


_ROWS = 2000


def _tc1_body(x_ref, w_ref, degp_ref, hs_ref, dinv_ref):
    d = lax.rsqrt(degp_ref[0, :, 0:1] + degp_ref[1, :, 0:1] + 1.0)
    dinv_ref[...] = d
    h = jnp.dot(x_ref[...], w_ref[...], preferred_element_type=jnp.float32)
    hs_ref[...] = h * d


def _tc2_body(aggp_ref, hs_ref, dinv_ref, b_ref, w_ref, out_ref):
    d = dinv_ref[...]
    t = (aggp_ref[0] + aggp_ref[1] + hs_ref[...]) * d + b_ref[...]
    h1 = jnp.maximum(t, 0.0)
    out_ref[...] = jnp.dot(h1, w_ref[...],
                           preferred_element_type=jnp.float32) * d


def _tc3_body(aggp_ref, hs_ref, dinv_ref, b_ref, out_ref):
    d = dinv_ref[...]
    out_ref[...] = (aggp_ref[0] + aggp_ref[1] + hs_ref[...]) * d + b_ref[...]


def _row_spec(width):
    return pl.BlockSpec((_ROWS, width), lambda i: (i, 0))


def _part_spec(width):
    return pl.BlockSpec((NC, _ROWS, width), lambda i: (0, i, 0))


def _full_spec(r, c):
    return pl.BlockSpec((r, c), lambda i: (0, 0))


_GRID = N_NODES // _ROWS


def _tc1(x, w1, degp):
    return pl.pallas_call(
        _tc1_body,
        grid=(_GRID,),
        in_specs=[_row_spec(D), _full_spec(D, D), _part_spec(D)],
        out_specs=[_row_spec(D), _row_spec(1)],
        out_shape=[jax.ShapeDtypeStruct((N_NODES, D), jnp.float32),
                   jax.ShapeDtypeStruct((N_NODES, 1), jnp.float32)],
    )(x, w1, degp)


def _tc2(aggp, hs, dinv, b, w2):
    return pl.pallas_call(
        _tc2_body,
        grid=(_GRID,),
        in_specs=[_part_spec(D), _row_spec(D), _row_spec(1),
                  _full_spec(1, D), _full_spec(D, D)],
        out_specs=_row_spec(D),
        out_shape=jax.ShapeDtypeStruct((N_NODES, D), jnp.float32),
    )(aggp, hs, dinv, b, w2)


def _tc3(aggp, hs, dinv, b):
    return pl.pallas_call(
        _tc3_body,
        grid=(_GRID,),
        in_specs=[_part_spec(D), _row_spec(D), _row_spec(1),
                  _full_spec(1, D)],
        out_specs=_row_spec(D),
        out_shape=jax.ShapeDtypeStruct((N_NODES, D), jnp.float32),
    )(aggp, hs, dinv, b)


_deg = _deg_kernel()
_scatter = _scatter_kernel()


@jax.jit
def kernel(x, edge_index, W1, b1, W2, b2):
    ei = edge_index.astype(jnp.int32)
    npad = E_PAD - N_EDGES
    pad_src = jnp.arange(npad, dtype=jnp.int32) % N_NODES
    pad_dst = N_NODES + (jnp.arange(npad, dtype=jnp.int32)
                         % (N_PAD - N_NODES))
    src = jnp.concatenate([ei[0], pad_src])
    dst = jnp.concatenate([ei[1], pad_dst])
    ones_deg = jnp.ones((K, D), jnp.float32)
    zeros_rows = jnp.zeros((RPT, D), jnp.float32)

    degp = _deg(dst, ones_deg, zeros_rows)[:, :N_NODES]
    hs1, dinv = _tc1(x, W1, degp)
    agg1 = _scatter(hs1, src, dst, zeros_rows)[:, :N_NODES]
    hs2 = _tc2(agg1, hs1, dinv, b1.reshape(1, D), W2)
    agg2 = _scatter(hs2, src, dst, zeros_rows)[:, :N_NODES]
    return _tc3(agg2, hs2, dinv, b2.reshape(1, D))


# speedup vs baseline: 4.0036x; 1.0643x over previous
import functools

import jax
import jax.numpy as jnp
from jax import lax
from jax.experimental import pallas as pl
from jax.experimental.pallas import tpu as pltpu
from jax.experimental.pallas import tpu_sc as plsc

N_NODES = 10000
N_EDGES = 320000
D = 128

NC = 2
NS = 16
NW = NC * NS
N_PAD = 10240
E_PAD = 327680
EPW = E_PAD // NW
K = 80
NCHUNK = EPW // K
RPT = N_PAD // NS


DEG_R = N_PAD // 128


def _deg_kernel():
    mesh = plsc.VectorSubcoreMesh(core_axis_name="c", subcore_axis_name="s")

    DNB = 8
    DPD = 4

    @functools.partial(
        pl.kernel,
        mesh=mesh,
        out_type=jax.ShapeDtypeStruct((NC, N_PAD, D), jnp.float32),
        scratch_types=(
            [pltpu.VMEM((K,), jnp.int32)] * DNB
            + [pltpu.VMEM((K, D), jnp.float32),
               pltpu.VMEM_SHARED((N_PAD, D), jnp.float32),
               pltpu.SemaphoreType.DMA((DNB,)),
               pltpu.SemaphoreType.DMA((DNB,))]
        ),
    )
    def k(dst_hbm, ones_hbm, zeros_hbm, out_hbm, *scr):
        dstv = scr[0:DNB]
        ones_v, acc, semId, semS = scr[DNB:]
        c = lax.axis_index("c")
        s = lax.axis_index("s")
        wid = c * NS + s
        r0 = s * RPT
        base0 = wid * EPW

        def idx_issue(j, b):
            off = pl.multiple_of(base0 + j * K, 8)
            pltpu.async_copy(dst_hbm.at[pl.ds(off, K)], dstv[b],
                             semId.at[b])

        def idx_wait(j, b):
            off = pl.multiple_of(base0 + j * K, 8)
            pltpu.make_async_copy(dst_hbm.at[pl.ds(off, K)], dstv[b],
                                  semId.at[b]).wait()

        def sca_wait(b):
            pltpu.make_async_copy(ones_v, acc.at[dstv[b]],
                                  semS.at[b]).wait()

        for j in range(DPD):
            idx_issue(j, j % DNB)
        pltpu.sync_copy(zeros_hbm, acc.at[pl.ds(r0, RPT)])
        pltpu.sync_copy(ones_hbm, ones_v)
        plsc.subcore_barrier()

        def body(g, carry):
            for b in range(DNB):
                i = g * DNB + b
                idx_wait(i, b)
                pltpu.async_copy(ones_v, acc.at[dstv[b]], semS.at[b],
                                 add=True)
                t = i + DPD
                bt = (b + DPD) % DNB

                @pl.when(t < NCHUNK)
                def _():
                    @pl.when(i >= DPD)
                    def _():
                        sca_wait(bt)

                    idx_issue(t, bt)
            return carry

        lax.fori_loop(0, NCHUNK // DNB, body, 0)
        for b in range(DNB):
            sca_wait(b)
        plsc.subcore_barrier()
        pltpu.sync_copy(acc.at[pl.ds(r0, RPT)],
                        out_hbm.at[c].at[pl.ds(r0, RPT)])

    return k


NB = 4
GD = 2


def _scatter_kernel():
    mesh = plsc.VectorSubcoreMesh(core_axis_name="c", subcore_axis_name="s")

    @functools.partial(
        pl.kernel,
        mesh=mesh,
        out_type=jax.ShapeDtypeStruct((NC, N_PAD, D), jnp.float32),
        scratch_types=(
            [pltpu.VMEM((K,), jnp.int32)] * NB
            + [pltpu.VMEM((K,), jnp.int32)] * NB
            + [pltpu.VMEM((K, D), jnp.float32)] * NB
            + [pltpu.VMEM_SHARED((N_PAD, D), jnp.float32),
               pltpu.SemaphoreType.DMA((NB,)),
               pltpu.SemaphoreType.DMA((NB,)),
               pltpu.SemaphoreType.DMA((NB,))]
        ),
    )
    def k(hs_hbm, src_hbm, dst_hbm, zeros_hbm, out_hbm, *scr):
        srcv = scr[0:NB]
        dstv = scr[NB:2 * NB]
        rows = scr[2 * NB:3 * NB]
        acc, semIs, semId, semG = scr[3 * NB:]
        c = lax.axis_index("c")
        s = lax.axis_index("s")
        wid = c * NS + s
        r0 = s * RPT
        base0 = wid * EPW

        def idx_issue(j, b):
            off = pl.multiple_of(base0 + j * K, 8)
            pltpu.async_copy(src_hbm.at[pl.ds(off, K)], srcv[b],
                             semIs.at[b])
            pltpu.async_copy(dst_hbm.at[pl.ds(off, K)], dstv[b],
                             semId.at[b])

        def idx_wait_src(j, b):
            off = pl.multiple_of(base0 + j * K, 8)
            pltpu.make_async_copy(src_hbm.at[pl.ds(off, K)], srcv[b],
                                  semIs.at[b]).wait()

        def idx_wait_dst(j, b):
            off = pl.multiple_of(base0 + j * K, 8)
            pltpu.make_async_copy(dst_hbm.at[pl.ds(off, K)], dstv[b],
                                  semId.at[b]).wait()

        for b in range(NB):
            idx_issue(b, b)
        pltpu.sync_copy(zeros_hbm, acc.at[pl.ds(r0, RPT)])
        for j in range(GD):
            idx_wait_src(j, j % NB)
            pltpu.async_copy(hs_hbm.at[srcv[j % NB]], rows[j % NB],
                             semG.at[j % NB])
        plsc.subcore_barrier()

        def body(g, carry):
            for b in range(NB):
                j = g * NB + b
                bg = (b + GD) % NB

                @pl.when(j + GD < NCHUNK)
                def _():
                    idx_wait_src(j + GD, bg)
                    pltpu.async_copy(hs_hbm.at[srcv[bg]], rows[bg],
                                     semG.at[bg])

                pltpu.make_async_copy(hs_hbm.at[srcv[b]], rows[b],
                                      semG.at[b]).wait()
                idx_wait_dst(j, b)
                pltpu.sync_copy(rows[b], acc.at[dstv[b]], add=True)

                @pl.when(j + NB < NCHUNK)
                def _():
                    idx_issue(j + NB, b)
            return carry

        lax.fori_loop(0, NCHUNK // NB, body, 0)
        plsc.subcore_barrier()
        pltpu.sync_copy(acc.at[pl.ds(r0, RPT)],
                        out_hbm.at[c].at[pl.ds(r0, RPT)])

    return k


_ROWS = 2000


def _tc1_body(x_ref, w_ref, degp_ref, hs_ref, dinv_ref):
    d = lax.rsqrt(degp_ref[0, :, 0:1] + degp_ref[1, :, 0:1] + 1.0)
    dinv_ref[...] = d
    h = jnp.dot(x_ref[...], w_ref[...], preferred_element_type=jnp.float32)
    hs_ref[...] = h * d


def _tc2_body(aggp_ref, hs_ref, dinv_ref, b_ref, w_ref, out_ref):
    d = dinv_ref[...]
    t = (aggp_ref[0] + aggp_ref[1] + hs_ref[...]) * d + b_ref[...]
    h1 = jnp.maximum(t, 0.0)
    out_ref[...] = jnp.dot(h1, w_ref[...],
                           preferred_element_type=jnp.float32) * d


def _tc3_body(aggp_ref, hs_ref, dinv_ref, b_ref, out_ref):
    d = dinv_ref[...]
    out_ref[...] = (aggp_ref[0] + aggp_ref[1] + hs_ref[...]) * d + b_ref[...]


def _row_spec(width):
    return pl.BlockSpec((_ROWS, width), lambda i: (i, 0))


def _part_spec(width):
    return pl.BlockSpec((NC, _ROWS, width), lambda i: (0, i, 0))


def _full_spec(r, c):
    return pl.BlockSpec((r, c), lambda i: (0, 0))


_GRID = N_NODES // _ROWS


def _tc1(x, w1, degp):
    return pl.pallas_call(
        _tc1_body,
        grid=(_GRID,),
        in_specs=[_row_spec(D), _full_spec(D, D), _part_spec(D)],
        out_specs=[_row_spec(D), _row_spec(1)],
        out_shape=[jax.ShapeDtypeStruct((N_NODES, D), jnp.float32),
                   jax.ShapeDtypeStruct((N_NODES, 1), jnp.float32)],
    )(x, w1, degp)


def _tc2(aggp, hs, dinv, b, w2):
    return pl.pallas_call(
        _tc2_body,
        grid=(_GRID,),
        in_specs=[_part_spec(D), _row_spec(D), _row_spec(1),
                  _full_spec(1, D), _full_spec(D, D)],
        out_specs=_row_spec(D),
        out_shape=jax.ShapeDtypeStruct((N_NODES, D), jnp.float32),
    )(aggp, hs, dinv, b, w2)


def _tc3(aggp, hs, dinv, b):
    return pl.pallas_call(
        _tc3_body,
        grid=(_GRID,),
        in_specs=[_part_spec(D), _row_spec(D), _row_spec(1),
                  _full_spec(1, D)],
        out_specs=_row_spec(D),
        out_shape=jax.ShapeDtypeStruct((N_NODES, D), jnp.float32),
    )(aggp, hs, dinv, b)


_deg = _deg_kernel()
_scatter = _scatter_kernel()


@jax.jit
def kernel(x, edge_index, W1, b1, W2, b2):
    ei = edge_index.astype(jnp.int32)
    npad = E_PAD - N_EDGES
    pad_src = jnp.arange(npad, dtype=jnp.int32) % N_NODES
    pad_dst = N_NODES + (jnp.arange(npad, dtype=jnp.int32)
                         % (N_PAD - N_NODES))
    src = jnp.concatenate([ei[0], pad_src])
    dst = jnp.concatenate([ei[1], pad_dst])
    ones_deg = jnp.ones((K, D), jnp.float32)
    zeros_rows = jnp.zeros((RPT, D), jnp.float32)

    degp = _deg(dst, ones_deg, zeros_rows)
    hs1, dinv = _tc1(x, W1, degp)
    agg1 = _scatter(hs1, src, dst, zeros_rows)
    hs2 = _tc2(agg1, hs1, dinv, b1.reshape(1, D), W2)
    agg2 = _scatter(hs2, src, dst, zeros_rows)
    return _tc3(agg2, hs2, dinv, b2.reshape(1, D))
